# baseline (speedup 1.0000x reference)
import functools

import jax
import jax.numpy as jnp
from jax import lax
from jax.experimental import pallas as pl
from jax.experimental.pallas import tpu as pltpu
from jax.experimental.pallas import tpu_sc as plsc

N = 10000
E = 320000
D = 128

NC = 2
NS = 16
NW = NC * NS
PER_TILE = E // NW
CS = 80
NCH = PER_TILE // CS
N_PAD = 10240
NROWS_T = N_PAD // NS
ZB = 128

_mesh = plsc.VectorSubcoreMesh(
    core_axis_name="c", subcore_axis_name="s", num_cores=NC, num_subcores=NS)


DH = D // 2


@functools.partial(
    pl.kernel,
    out_type=[jax.ShapeDtypeStruct((E, DH), jnp.int32),
              jax.ShapeDtypeStruct((E, DH), jnp.int32)],
    mesh=_mesh,
    scratch_types=[
        pltpu.VMEM((PER_TILE,), jnp.int32),
        pltpu.VMEM((PER_TILE,), jnp.int32),
        pltpu.VMEM((CS, DH), jnp.int32),
        pltpu.VMEM((CS, DH), jnp.int32),
    ],
    compiler_params=pltpu.CompilerParams(use_tc_tiling_on_sc=False),
)
def _sc_gather(x_hbm, src_hbm, dst_hbm, gs_hbm, gd_hbm,
               idx_s, idx_d, buf_s, buf_d):
    cid = lax.axis_index("c")
    sid = lax.axis_index("s")
    wid = cid * NS + sid
    base = wid * PER_TILE
    pltpu.sync_copy(src_hbm.at[pl.ds(base, PER_TILE)], idx_s)
    pltpu.sync_copy(dst_hbm.at[pl.ds(base, PER_TILE)], idx_d)

    def body(j, carry):
        off = j * CS
        pltpu.sync_copy(x_hbm.at[idx_s.at[pl.ds(off, CS)]], buf_s)
        pltpu.sync_copy(buf_s, gs_hbm.at[pl.ds(base + off, CS)])
        pltpu.sync_copy(x_hbm.at[idx_d.at[pl.ds(off, CS)]], buf_d)
        pltpu.sync_copy(buf_d, gd_hbm.at[pl.ds(base + off, CS)])
        return carry

    lax.fori_loop(0, NCH, body, 0)


@functools.partial(
    pl.kernel,
    out_type=jax.ShapeDtypeStruct((NC, N_PAD, D), jnp.float32),
    mesh=_mesh,
    scratch_types=[
        pltpu.VMEM((NCH, CS), jnp.int32),
        pltpu.VMEM((CS, D), jnp.float32),
        pltpu.VMEM((ZB, D), jnp.float32),
        pltpu.VMEM_SHARED((N_PAD, D), jnp.float32),
    ],
)
def _sc_scatter(msg_hbm, dst3_hbm, out_hbm, idx_all, rows, zbuf, agg_sh):
    cid = lax.axis_index("c")
    sid = lax.axis_index("s")
    wid = cid * NS + sid

    def zb(t, carry):
        i = t // (D // 16)
        k = t % (D // 16)
        zbuf[i, pl.ds(k * 16, 16)] = jnp.zeros((16,), jnp.float32)
        return carry

    lax.fori_loop(0, ZB * (D // 16), zb, 0)
    row0 = sid * NROWS_T
    for m in range(NROWS_T // ZB):
        pltpu.sync_copy(zbuf, agg_sh.at[pl.ds(row0 + m * ZB, ZB)])
    plsc.subcore_barrier()

    ebase = wid * PER_TILE
    pltpu.sync_copy(dst3_hbm.at[wid], idx_all)

    def body(j, carry):
        pltpu.sync_copy(msg_hbm.at[pl.ds(ebase + j * CS, CS)], rows)
        pltpu.sync_copy(rows, agg_sh.at[idx_all.at[j]], add=True)
        return carry

    lax.fori_loop(0, NCH, body, 0)
    plsc.subcore_barrier()

    for m in range(NROWS_T // ZB):
        r = row0 + m * ZB
        pltpu.sync_copy(agg_sh.at[pl.ds(r, ZB)], zbuf)
        pltpu.sync_copy(zbuf, out_hbm.at[cid].at[pl.ds(r, ZB)])


BE = 2560


def _edge_body(gs, gd, wpe, wce, m1p, m1c, m1d, bpe, bce, bm1, w2, bm2, msg):
    xs = gs[...]
    xd = gd[...]
    t1 = jnp.dot(xs, wpe[...], preferred_element_type=jnp.float32) + bpe[...]
    t1 = jnp.where(t1 >= 0, t1, 0.01 * t1).astype(jnp.bfloat16)
    t2 = jnp.dot(xd, wce[...], preferred_element_type=jnp.float32) + bce[...]
    t2 = jnp.where(t2 >= 0, t2, 0.01 * t2).astype(jnp.bfloat16)
    pre = (jnp.dot(t1, m1p[...], preferred_element_type=jnp.float32)
           + jnp.dot(t2, m1c[...], preferred_element_type=jnp.float32)
           + jnp.dot(jnp.abs(xs - xd), m1d[...],
                     preferred_element_type=jnp.float32)
           + bm1[...])
    h = jnp.maximum(pre, 0.0)
    z = jnp.sum(h * w2[...], axis=1, keepdims=True) + bm2[...]
    ew = 1.0 / (1.0 + jnp.exp(-z))
    msg[...] = xs.astype(jnp.float32) * ew


def _edge_mlp(gs, gd, wpe, wce, m1p, m1c, m1d, bpe, bce, bm1, w2, bm2):
    full = lambda shp: pl.BlockSpec(shp, lambda i: (0,) * len(shp))
    return pl.pallas_call(
        _edge_body,
        grid=(E // BE,),
        in_specs=[
            pl.BlockSpec((BE, D), lambda i: (i, 0)),
            pl.BlockSpec((BE, D), lambda i: (i, 0)),
            full((D, D)), full((D, D)), full((D, D)), full((D, D)),
            full((D, D)),
            full((1, D)), full((1, D)), full((1, D)), full((1, D)),
            full((1, 1)),
        ],
        out_specs=pl.BlockSpec((BE, D), lambda i: (i, 0)),
        out_shape=jax.ShapeDtypeStruct((E, D), jnp.float32),
    )(gs, gd, wpe, wce, m1p, m1c, m1d, bpe, bce, bm1, w2, bm2)


BN = 2000


def _post_body(a0, a1, xb, wrel, wroot, brel, out):
    agg = a0[...] + a1[...]
    out[...] = (jnp.dot(agg, wrel[...], preferred_element_type=jnp.float32)
                + jnp.dot(xb[...], wroot[...],
                          preferred_element_type=jnp.float32)
                + brel[...])


def _post(a0, a1, x, wrel, wroot, brel):
    full = lambda shp: pl.BlockSpec(shp, lambda i: (0,) * len(shp))
    return pl.pallas_call(
        _post_body,
        grid=(N // BN,),
        in_specs=[
            pl.BlockSpec((BN, D), lambda i: (i, 0)),
            pl.BlockSpec((BN, D), lambda i: (i, 0)),
            pl.BlockSpec((BN, D), lambda i: (i, 0)),
            full((D, D)), full((D, D)), full((1, D)),
        ],
        out_specs=pl.BlockSpec((BN, D), lambda i: (i, 0)),
        out_shape=jax.ShapeDtypeStruct((N, D), jnp.float32),
    )(a0, a1, x, wrel, wroot, brel)


def kernel(x, edge_index, W_pe, b_pe, W_ce, b_ce, W_m1, b_m1, W_m2, b_m2,
           W_rel, b_rel, W_root):
    src = edge_index[0]
    dst = edge_index[1]
    dst3 = dst.reshape(NW, NCH, CS)

    bf = jnp.bfloat16
    x_i = lax.bitcast_convert_type(
        x.astype(bf).reshape(N, DH, 2), jnp.int32)
    gs_i, gd_i = _sc_gather(x_i, src, dst)
    gs = lax.bitcast_convert_type(gs_i, bf).reshape(E, D)
    gd = lax.bitcast_convert_type(gd_i, bf).reshape(E, D)

    msg = _edge_mlp(
        gs, gd, W_pe.astype(bf), W_ce.astype(bf),
        W_m1[:D].astype(bf), W_m1[D:2 * D].astype(bf),
        W_m1[2 * D:].astype(bf),
        b_pe.reshape(1, D), b_ce.reshape(1, D), b_m1.reshape(1, D),
        W_m2.reshape(1, D), b_m2.reshape(1, 1))

    aggs = _sc_scatter(msg, dst3)

    return _post(aggs[0], aggs[1], x, W_rel, W_root, b_rel.reshape(1, D))


# speedup vs baseline: 2.9601x; 2.9601x over previous
import functools

import jax
import jax.numpy as jnp
from jax import lax
from jax.experimental import pallas as pl
from jax.experimental.pallas import tpu as pltpu
from jax.experimental.pallas import tpu_sc as plsc

N = 10000
E = 320000
D = 128

NC = 2
NS = 16
NW = NC * NS
PER_TILE = E // NW
CS = 80
NCH = PER_TILE // CS
N_PAD = 10240
NROWS_T = N_PAD // NS
ZB = 128

_mesh = plsc.VectorSubcoreMesh(
    core_axis_name="c", subcore_axis_name="s", num_cores=NC, num_subcores=NS)


def _pack(xbf, abf):
    xu = lax.bitcast_convert_type(xbf, jnp.uint16).astype(jnp.uint32)
    au = lax.bitcast_convert_type(abf, jnp.uint16).astype(jnp.uint32)
    return lax.bitcast_convert_type(xu | (au << 16), jnp.int32)


def _unpack(w):
    wu = lax.bitcast_convert_type(w, jnp.uint32)
    lo = lax.bitcast_convert_type((wu & 0xFFFF).astype(jnp.uint16),
                                  jnp.bfloat16)
    hi = lax.bitcast_convert_type((wu >> 16).astype(jnp.uint16),
                                  jnp.bfloat16)
    return lo, hi


BP = 2000


def _pre_body(xb, wpe, wce, m1p, m1c, bpe, bce, ts, td):
    xv = xb[...]
    t1 = jnp.dot(xv, wpe[...], preferred_element_type=jnp.float32) + bpe[...]
    t1 = jnp.where(t1 >= 0, t1, 0.01 * t1)
    a = jnp.dot(t1, m1p[...], preferred_element_type=jnp.float32)
    t2 = jnp.dot(xv, wce[...], preferred_element_type=jnp.float32) + bce[...]
    t2 = jnp.where(t2 >= 0, t2, 0.01 * t2)
    b = jnp.dot(t2, m1c[...], preferred_element_type=jnp.float32)
    xbf = xv.astype(jnp.bfloat16)
    ts[...] = _pack(xbf, a.astype(jnp.bfloat16))
    td[...] = _pack(xbf, b.astype(jnp.bfloat16))


def _pre(x, wpe, wce, m1p, m1c, bpe, bce):
    full = lambda shp: pl.BlockSpec(shp, lambda i: (0,) * len(shp))
    return pl.pallas_call(
        _pre_body,
        grid=(N // BP,),
        in_specs=[
            pl.BlockSpec((BP, D), lambda i: (i, 0)),
            full((D, D)), full((D, D)), full((D, D)), full((D, D)),
            full((1, D)), full((1, D)),
        ],
        out_specs=[pl.BlockSpec((BP, D), lambda i: (i, 0)),
                   pl.BlockSpec((BP, D), lambda i: (i, 0))],
        out_shape=[jax.ShapeDtypeStruct((N, D), jnp.int32),
                   jax.ShapeDtypeStruct((N, D), jnp.int32)],
    )(x, wpe, wce, m1p, m1c, bpe, bce)


@functools.partial(
    pl.kernel,
    out_type=[jax.ShapeDtypeStruct((E, D), jnp.int32),
              jax.ShapeDtypeStruct((E, D), jnp.int32)],
    mesh=_mesh,
    scratch_types=[
        pltpu.VMEM((PER_TILE,), jnp.int32),
        pltpu.VMEM((PER_TILE,), jnp.int32),
        pltpu.VMEM((CS, D), jnp.int32),
        pltpu.VMEM((CS, D), jnp.int32),
    ],
)
def _sc_gather(ts_hbm, td_hbm, src_hbm, dst_hbm, gs_hbm, gd_hbm,
               idx_s, idx_d, buf_s, buf_d):
    cid = lax.axis_index("c")
    sid = lax.axis_index("s")
    wid = cid * NS + sid
    base = wid * PER_TILE
    pltpu.sync_copy(src_hbm.at[pl.ds(base, PER_TILE)], idx_s)
    pltpu.sync_copy(dst_hbm.at[pl.ds(base, PER_TILE)], idx_d)

    def body(j, carry):
        off = j * CS
        pltpu.sync_copy(ts_hbm.at[idx_s.at[pl.ds(off, CS)]], buf_s)
        pltpu.sync_copy(buf_s, gs_hbm.at[pl.ds(base + off, CS)])
        pltpu.sync_copy(td_hbm.at[idx_d.at[pl.ds(off, CS)]], buf_d)
        pltpu.sync_copy(buf_d, gd_hbm.at[pl.ds(base + off, CS)])
        return carry

    lax.fori_loop(0, NCH, body, 0)


@functools.partial(
    pl.kernel,
    out_type=jax.ShapeDtypeStruct((NC, N_PAD, D), jnp.float32),
    mesh=_mesh,
    scratch_types=[
        pltpu.VMEM((NCH, CS), jnp.int32),
        pltpu.VMEM((CS, D), jnp.float32),
        pltpu.VMEM((ZB, D), jnp.float32),
        pltpu.VMEM_SHARED((N_PAD, D), jnp.float32),
    ],
)
def _sc_scatter(msg_hbm, dst3_hbm, out_hbm, idx_all, rows, zbuf, agg_sh):
    cid = lax.axis_index("c")
    sid = lax.axis_index("s")
    wid = cid * NS + sid

    def zb(t, carry):
        i = t // (D // 16)
        k = t % (D // 16)
        zbuf[i, pl.ds(k * 16, 16)] = jnp.zeros((16,), jnp.float32)
        return carry

    lax.fori_loop(0, ZB * (D // 16), zb, 0)
    row0 = sid * NROWS_T
    for m in range(NROWS_T // ZB):
        pltpu.sync_copy(zbuf, agg_sh.at[pl.ds(row0 + m * ZB, ZB)])
    plsc.subcore_barrier()

    ebase = wid * PER_TILE
    pltpu.sync_copy(dst3_hbm.at[wid], idx_all)

    def body(j, carry):
        pltpu.sync_copy(msg_hbm.at[pl.ds(ebase + j * CS, CS)], rows)
        pltpu.sync_copy(rows, agg_sh.at[idx_all.at[j]], add=True)
        return carry

    lax.fori_loop(0, NCH, body, 0)
    plsc.subcore_barrier()

    for m in range(NROWS_T // ZB):
        r = row0 + m * ZB
        pltpu.sync_copy(agg_sh.at[pl.ds(r, ZB)], zbuf)
        pltpu.sync_copy(zbuf, out_hbm.at[cid].at[pl.ds(r, ZB)])


BE = 2560


def _edge_body(gs, gd, m1d, bm1, w2, bm2, msg):
    xs, a_s = _unpack(gs[...])
    xd, b_d = _unpack(gd[...])
    pre = (jnp.dot(jnp.abs(xs - xd), m1d[...],
                   preferred_element_type=jnp.float32)
           + a_s.astype(jnp.float32) + b_d.astype(jnp.float32) + bm1[...])
    h = jnp.maximum(pre, 0.0)
    z = jnp.sum(h * w2[...], axis=1, keepdims=True) + bm2[...]
    ew = 1.0 / (1.0 + jnp.exp(-z))
    msg[...] = xs.astype(jnp.float32) * ew


def _edge_mlp(gs, gd, m1d, bm1, w2, bm2):
    full = lambda shp: pl.BlockSpec(shp, lambda i: (0,) * len(shp))
    return pl.pallas_call(
        _edge_body,
        grid=(E // BE,),
        in_specs=[
            pl.BlockSpec((BE, D), lambda i: (i, 0)),
            pl.BlockSpec((BE, D), lambda i: (i, 0)),
            full((D, D)),
            full((1, D)), full((1, D)), full((1, 1)),
        ],
        out_specs=pl.BlockSpec((BE, D), lambda i: (i, 0)),
        out_shape=jax.ShapeDtypeStruct((E, D), jnp.float32),
    )(gs, gd, m1d, bm1, w2, bm2)


BN = 2000


def _post_body(a0, a1, xb, wrel, wroot, brel, out):
    agg = a0[...] + a1[...]
    out[...] = (jnp.dot(agg, wrel[...], preferred_element_type=jnp.float32)
                + jnp.dot(xb[...], wroot[...],
                          preferred_element_type=jnp.float32)
                + brel[...])


def _post(a0, a1, x, wrel, wroot, brel):
    full = lambda shp: pl.BlockSpec(shp, lambda i: (0,) * len(shp))
    return pl.pallas_call(
        _post_body,
        grid=(N // BN,),
        in_specs=[
            pl.BlockSpec((BN, D), lambda i: (i, 0)),
            pl.BlockSpec((BN, D), lambda i: (i, 0)),
            pl.BlockSpec((BN, D), lambda i: (i, 0)),
            full((D, D)), full((D, D)), full((1, D)),
        ],
        out_specs=pl.BlockSpec((BN, D), lambda i: (i, 0)),
        out_shape=jax.ShapeDtypeStruct((N, D), jnp.float32),
    )(a0, a1, x, wrel, wroot, brel)


def kernel(x, edge_index, W_pe, b_pe, W_ce, b_ce, W_m1, b_m1, W_m2, b_m2,
           W_rel, b_rel, W_root):
    src = edge_index[0]
    dst = edge_index[1]
    dst3 = dst.reshape(NW, NCH, CS)
    bf = jnp.bfloat16

    ts, td = _pre(x, W_pe, W_ce, W_m1[:D], W_m1[D:2 * D],
                  b_pe.reshape(1, D), b_ce.reshape(1, D))

    gs, gd = _sc_gather(ts, td, src, dst)

    msg = _edge_mlp(gs, gd, W_m1[2 * D:].astype(bf),
                    b_m1.reshape(1, D), W_m2.reshape(1, D),
                    b_m2.reshape(1, 1))

    aggs = _sc_scatter(msg, dst3)

    return _post(aggs[0], aggs[1], x, W_rel, W_root, b_rel.reshape(1, D))


# speedup vs baseline: 2.9722x; 1.0041x over previous
import functools

import jax
import jax.numpy as jnp
from jax import lax
from jax.experimental import pallas as pl
from jax.experimental.pallas import tpu as pltpu
from jax.experimental.pallas import tpu_sc as plsc

N = 10000
E = 320000
D = 128

NC = 2
NS = 16
NW = NC * NS
PER_TILE = E // NW
CS = 80
NCH = PER_TILE // CS
N_PAD = 10240
NROWS_T = N_PAD // NS
ZB = 128

_mesh = plsc.VectorSubcoreMesh(
    core_axis_name="c", subcore_axis_name="s", num_cores=NC, num_subcores=NS)


def _pack(xf, af):
    xu = lax.bitcast_convert_type(xf, jnp.uint32) >> 16
    au = lax.bitcast_convert_type(af, jnp.uint32) & jnp.uint32(0xFFFF0000)
    return lax.bitcast_convert_type(xu | au, jnp.int32)


def _unpack(w):
    wu = lax.bitcast_convert_type(w, jnp.uint32)
    lo = lax.bitcast_convert_type(wu << 16, jnp.float32)
    hi = lax.bitcast_convert_type(wu & jnp.uint32(0xFFFF0000), jnp.float32)
    return lo, hi


BP = 2000


def _pre_body(xb, wpe, wce, m1p, m1c, bpe, bce, ts, td):
    xv = xb[...]
    t1 = jnp.dot(xv, wpe[...], preferred_element_type=jnp.float32) + bpe[...]
    t1 = jnp.where(t1 >= 0, t1, 0.01 * t1)
    a = jnp.dot(t1, m1p[...], preferred_element_type=jnp.float32)
    t2 = jnp.dot(xv, wce[...], preferred_element_type=jnp.float32) + bce[...]
    t2 = jnp.where(t2 >= 0, t2, 0.01 * t2)
    b = jnp.dot(t2, m1c[...], preferred_element_type=jnp.float32)
    xr = xv.astype(jnp.bfloat16).astype(jnp.float32)
    ts[...] = _pack(xr, a.astype(jnp.bfloat16).astype(jnp.float32))
    td[...] = _pack(xr, b.astype(jnp.bfloat16).astype(jnp.float32))


def _pre(x, wpe, wce, m1p, m1c, bpe, bce):
    full = lambda shp: pl.BlockSpec(shp, lambda i: (0,) * len(shp))
    return pl.pallas_call(
        _pre_body,
        grid=(N // BP,),
        in_specs=[
            pl.BlockSpec((BP, D), lambda i: (i, 0)),
            full((D, D)), full((D, D)), full((D, D)), full((D, D)),
            full((1, D)), full((1, D)),
        ],
        out_specs=[pl.BlockSpec((BP, D), lambda i: (i, 0)),
                   pl.BlockSpec((BP, D), lambda i: (i, 0))],
        out_shape=[jax.ShapeDtypeStruct((N, D), jnp.int32),
                   jax.ShapeDtypeStruct((N, D), jnp.int32)],
    )(x, wpe, wce, m1p, m1c, bpe, bce)


@functools.partial(
    pl.kernel,
    out_type=[jax.ShapeDtypeStruct((E, D), jnp.int32),
              jax.ShapeDtypeStruct((E, D), jnp.int32)],
    mesh=_mesh,
    scratch_types=[
        pltpu.VMEM((PER_TILE,), jnp.int32),
        pltpu.VMEM((PER_TILE,), jnp.int32),
        pltpu.VMEM((CS, D), jnp.int32),
        pltpu.VMEM((CS, D), jnp.int32),
    ],
)
def _sc_gather(ts_hbm, td_hbm, src_hbm, dst_hbm, gs_hbm, gd_hbm,
               idx_s, idx_d, buf_s, buf_d):
    cid = lax.axis_index("c")
    sid = lax.axis_index("s")
    wid = cid * NS + sid
    base = wid * PER_TILE
    pltpu.sync_copy(src_hbm.at[pl.ds(base, PER_TILE)], idx_s)
    pltpu.sync_copy(dst_hbm.at[pl.ds(base, PER_TILE)], idx_d)

    def body(j, carry):
        off = j * CS
        pltpu.sync_copy(ts_hbm.at[idx_s.at[pl.ds(off, CS)]], buf_s)
        pltpu.sync_copy(buf_s, gs_hbm.at[pl.ds(base + off, CS)])
        pltpu.sync_copy(td_hbm.at[idx_d.at[pl.ds(off, CS)]], buf_d)
        pltpu.sync_copy(buf_d, gd_hbm.at[pl.ds(base + off, CS)])
        return carry

    lax.fori_loop(0, NCH, body, 0)


@functools.partial(
    pl.kernel,
    out_type=jax.ShapeDtypeStruct((NC, N_PAD, D), jnp.float32),
    mesh=_mesh,
    scratch_types=[
        pltpu.VMEM((NCH, CS), jnp.int32),
        pltpu.VMEM((CS, D), jnp.float32),
        pltpu.VMEM((ZB, D), jnp.float32),
        pltpu.VMEM_SHARED((N_PAD, D), jnp.float32),
    ],
)
def _sc_scatter(msg_hbm, dst3_hbm, out_hbm, idx_all, rows, zbuf, agg_sh):
    cid = lax.axis_index("c")
    sid = lax.axis_index("s")
    wid = cid * NS + sid

    def zb(t, carry):
        i = t // (D // 16)
        k = t % (D // 16)
        zbuf[i, pl.ds(k * 16, 16)] = jnp.zeros((16,), jnp.float32)
        return carry

    lax.fori_loop(0, ZB * (D // 16), zb, 0)
    row0 = sid * NROWS_T
    for m in range(NROWS_T // ZB):
        pltpu.sync_copy(zbuf, agg_sh.at[pl.ds(row0 + m * ZB, ZB)])
    plsc.subcore_barrier()

    ebase = wid * PER_TILE
    pltpu.sync_copy(dst3_hbm.at[wid], idx_all)

    def body(j, carry):
        pltpu.sync_copy(msg_hbm.at[pl.ds(ebase + j * CS, CS)], rows)
        pltpu.sync_copy(rows, agg_sh.at[idx_all.at[j]], add=True)
        return carry

    lax.fori_loop(0, NCH, body, 0)
    plsc.subcore_barrier()

    for m in range(NROWS_T // ZB):
        r = row0 + m * ZB
        pltpu.sync_copy(agg_sh.at[pl.ds(r, ZB)], zbuf)
        pltpu.sync_copy(zbuf, out_hbm.at[cid].at[pl.ds(r, ZB)])


BE = 2560


def _edge_body(gs, gd, m1d, bm1, w2, bm2, msg):
    xs, a_s = _unpack(gs[...])
    xd, b_d = _unpack(gd[...])
    diff = jnp.abs(xs - xd).astype(jnp.bfloat16)
    pre = (jnp.dot(diff, m1d[...], preferred_element_type=jnp.float32)
           + a_s + b_d + bm1[...])
    h = jnp.maximum(pre, 0.0)
    z = jnp.sum(h * w2[...], axis=1, keepdims=True) + bm2[...]
    ew = 1.0 / (1.0 + jnp.exp(-z))
    msg[...] = xs * ew


def _edge_mlp(gs, gd, m1d, bm1, w2, bm2):
    full = lambda shp: pl.BlockSpec(shp, lambda i: (0,) * len(shp))
    return pl.pallas_call(
        _edge_body,
        grid=(E // BE,),
        in_specs=[
            pl.BlockSpec((BE, D), lambda i: (i, 0)),
            pl.BlockSpec((BE, D), lambda i: (i, 0)),
            full((D, D)),
            full((1, D)), full((1, D)), full((1, 1)),
        ],
        out_specs=pl.BlockSpec((BE, D), lambda i: (i, 0)),
        out_shape=jax.ShapeDtypeStruct((E, D), jnp.float32),
    )(gs, gd, m1d, bm1, w2, bm2)


BN = 2000


def _post_body(a0, a1, xb, wrel, wroot, brel, out):
    agg = a0[...] + a1[...]
    out[...] = (jnp.dot(agg, wrel[...], preferred_element_type=jnp.float32)
                + jnp.dot(xb[...], wroot[...],
                          preferred_element_type=jnp.float32)
                + brel[...])


def _post(a0, a1, x, wrel, wroot, brel):
    full = lambda shp: pl.BlockSpec(shp, lambda i: (0,) * len(shp))
    return pl.pallas_call(
        _post_body,
        grid=(N // BN,),
        in_specs=[
            pl.BlockSpec((BN, D), lambda i: (i, 0)),
            pl.BlockSpec((BN, D), lambda i: (i, 0)),
            pl.BlockSpec((BN, D), lambda i: (i, 0)),
            full((D, D)), full((D, D)), full((1, D)),
        ],
        out_specs=pl.BlockSpec((BN, D), lambda i: (i, 0)),
        out_shape=jax.ShapeDtypeStruct((N, D), jnp.float32),
    )(a0, a1, x, wrel, wroot, brel)


def kernel(x, edge_index, W_pe, b_pe, W_ce, b_ce, W_m1, b_m1, W_m2, b_m2,
           W_rel, b_rel, W_root):
    src = edge_index[0]
    dst = edge_index[1]
    dst3 = dst.reshape(NW, NCH, CS)
    bf = jnp.bfloat16

    ts, td = _pre(x, W_pe, W_ce, W_m1[:D], W_m1[D:2 * D],
                  b_pe.reshape(1, D), b_ce.reshape(1, D))

    gs, gd = _sc_gather(ts, td, src, dst)

    msg = _edge_mlp(gs, gd, W_m1[2 * D:].astype(bf),
                    b_m1.reshape(1, D), W_m2.reshape(1, D),
                    b_m2.reshape(1, 1))

    aggs = _sc_scatter(msg, dst3)

    return _post(aggs[0], aggs[1], x, W_rel, W_root, b_rel.reshape(1, D))


# speedup vs baseline: 4.0360x; 1.3579x over previous
import functools

import jax
import jax.numpy as jnp
from jax import lax
from jax.experimental import pallas as pl
from jax.experimental.pallas import tpu as pltpu
from jax.experimental.pallas import tpu_sc as plsc

N = 10000
E = 320000
D = 128

NC = 2
NS = 16
NW = NC * NS
PER_TILE = E // NW
CS = 40
NCH = PER_TILE // CS
GK = 5
NG = NCH // GK
N_PAD = 10240
NROWS_T = N_PAD // NS
ZB = 128

_mesh = plsc.VectorSubcoreMesh(
    core_axis_name="c", subcore_axis_name="s", num_cores=NC, num_subcores=NS)


def _pack(xf, af):
    xu = lax.bitcast_convert_type(xf, jnp.uint32) >> 16
    au = lax.bitcast_convert_type(af, jnp.uint32) & jnp.uint32(0xFFFF0000)
    return lax.bitcast_convert_type(xu | au, jnp.int32)


def _unpack(w):
    wu = lax.bitcast_convert_type(w, jnp.uint32)
    lo = lax.bitcast_convert_type(wu << 16, jnp.float32)
    hi = lax.bitcast_convert_type(wu & jnp.uint32(0xFFFF0000), jnp.float32)
    return lo, hi


BP = 2000


def _pre_body(xb, wpe, wce, m1p, m1c, bpe, bce, ts, td):
    xv = xb[...]
    t1 = jnp.dot(xv, wpe[...], preferred_element_type=jnp.float32) + bpe[...]
    t1 = jnp.where(t1 >= 0, t1, 0.01 * t1)
    a = jnp.dot(t1, m1p[...], preferred_element_type=jnp.float32)
    t2 = jnp.dot(xv, wce[...], preferred_element_type=jnp.float32) + bce[...]
    t2 = jnp.where(t2 >= 0, t2, 0.01 * t2)
    b = jnp.dot(t2, m1c[...], preferred_element_type=jnp.float32)
    xr = xv.astype(jnp.bfloat16).astype(jnp.float32)
    ts[...] = _pack(xr, a.astype(jnp.bfloat16).astype(jnp.float32))
    td[...] = _pack(xr, b.astype(jnp.bfloat16).astype(jnp.float32))


def _pre(x, wpe, wce, m1p, m1c, bpe, bce):
    full = lambda shp: pl.BlockSpec(shp, lambda i: (0,) * len(shp))
    return pl.pallas_call(
        _pre_body,
        grid=(N // BP,),
        in_specs=[
            pl.BlockSpec((BP, D), lambda i: (i, 0)),
            full((D, D)), full((D, D)), full((D, D)), full((D, D)),
            full((1, D)), full((1, D)),
        ],
        out_specs=[pl.BlockSpec((BP, D), lambda i: (i, 0)),
                   pl.BlockSpec((BP, D), lambda i: (i, 0))],
        out_shape=[jax.ShapeDtypeStruct((N, D), jnp.int32),
                   jax.ShapeDtypeStruct((N, D), jnp.int32)],
    )(x, wpe, wce, m1p, m1c, bpe, bce)


@functools.partial(
    pl.kernel,
    out_type=[jax.ShapeDtypeStruct((E, D), jnp.int32),
              jax.ShapeDtypeStruct((E, D), jnp.int32)],
    mesh=_mesh,
    scratch_types=[
        pltpu.VMEM((PER_TILE,), jnp.int32),
        pltpu.VMEM((PER_TILE,), jnp.int32),
        pltpu.VMEM((2 * GK, CS, D), jnp.int32),
        pltpu.VMEM((2 * GK, CS, D), jnp.int32),
    ] + [pltpu.SemaphoreType.DMA] * 8,
)
def _sc_gather(ts_hbm, td_hbm, src_hbm, dst_hbm, gs_hbm, gd_hbm,
               idx_s, idx_d, buf_s, buf_d,
               sg_s0, sg_s1, sg_d0, sg_d1, sw_s0, sw_s1, sw_d0, sw_d1):
    cid = lax.axis_index("c")
    sid = lax.axis_index("s")
    wid = cid * NS + sid
    base = wid * PER_TILE
    pltpu.sync_copy(src_hbm.at[pl.ds(base, PER_TILE)], idx_s)
    pltpu.sync_copy(dst_hbm.at[pl.ds(base, PER_TILE)], idx_d)
    sg = ((sg_s0, sg_d0), (sg_s1, sg_d1))
    sw = ((sw_s0, sw_d0), (sw_s1, sw_d1))

    def issue_gathers(g, p):
        for i in range(GK):
            off = (g * GK + i) * CS
            k = p * GK + i
            pltpu.async_copy(ts_hbm.at[idx_s.at[pl.ds(off, CS)]],
                             buf_s.at[k], sg[p][0])
            pltpu.async_copy(td_hbm.at[idx_d.at[pl.ds(off, CS)]],
                             buf_d.at[k], sg[p][1])

    def drain_gathers(g, p):
        for i in range(GK):
            off = (g * GK + i) * CS
            k = p * GK + i
            pltpu.make_async_copy(ts_hbm.at[idx_s.at[pl.ds(off, CS)]],
                                  buf_s.at[k], sg[p][0]).wait()
            pltpu.make_async_copy(td_hbm.at[idx_d.at[pl.ds(off, CS)]],
                                  buf_d.at[k], sg[p][1]).wait()

    def issue_writes(g, p):
        for i in range(GK):
            off = (g * GK + i) * CS
            k = p * GK + i
            pltpu.async_copy(buf_s.at[k], gs_hbm.at[pl.ds(base + off, CS)],
                             sw[p][0])
            pltpu.async_copy(buf_d.at[k], gd_hbm.at[pl.ds(base + off, CS)],
                             sw[p][1])

    def drain_writes(g, p):
        for i in range(GK):
            off = (g * GK + i) * CS
            k = p * GK + i
            pltpu.make_async_copy(buf_s.at[k],
                                  gs_hbm.at[pl.ds(base + off, CS)],
                                  sw[p][0]).wait()
            pltpu.make_async_copy(buf_d.at[k],
                                  gd_hbm.at[pl.ds(base + off, CS)],
                                  sw[p][1]).wait()

    issue_gathers(0, 0)

    def body(gg, carry):
        for p in (0, 1):
            g = 2 * gg + p

            @pl.when(g >= 2)
            def _():
                drain_writes(g - 2, p)

            @pl.when(g >= 1)
            def _():
                issue_gathers(g, p)

            @pl.when(g >= 1)
            def _():
                drain_gathers(g - 1, 1 - p)
                issue_writes(g - 1, 1 - p)

        return carry

    lax.fori_loop(0, NG // 2, body, 0)
    drain_gathers(NG - 1, 1)
    issue_writes(NG - 1, 1)
    drain_writes(NG - 2, 0)
    drain_writes(NG - 1, 1)


GK2 = 1
NB2 = 2 * GK2 * CS
NG2 = NCH // GK2


@functools.partial(
    pl.kernel,
    out_type=jax.ShapeDtypeStruct((NC, N_PAD, D), jnp.float32),
    mesh=_mesh,
    scratch_types=[
        pltpu.VMEM((NCH, CS), jnp.int32),
        pltpu.VMEM((NB2, D), jnp.float32),
        pltpu.VMEM_SHARED((N_PAD, D), jnp.float32),
    ] + [pltpu.SemaphoreType.DMA] * 4,
)
def _sc_scatter(msg_hbm, dst3_hbm, out_hbm, idx_all, rows, agg_sh,
                sr0, sr1, sa0, sa1):
    cid = lax.axis_index("c")
    sid = lax.axis_index("s")
    wid = cid * NS + sid

    def zb(t, carry):
        i = t // (D // 16)
        k = t % (D // 16)
        rows[i, pl.ds(k * 16, 16)] = jnp.zeros((16,), jnp.float32)
        return carry

    lax.fori_loop(0, NB2 * (D // 16), zb, 0)
    row0 = sid * NROWS_T
    for m in range(NROWS_T // NB2):
        pltpu.sync_copy(rows, agg_sh.at[pl.ds(row0 + m * NB2, NB2)])
    plsc.subcore_barrier()

    ebase = wid * PER_TILE
    pltpu.sync_copy(dst3_hbm.at[wid], idx_all)
    sr = (sr0, sr1)
    sa = (sa0, sa1)

    def issue_reads(g, p):
        for i in range(GK2):
            off = (g * GK2 + i) * CS
            k = (p * GK2 + i) * CS
            pltpu.async_copy(msg_hbm.at[pl.ds(ebase + off, CS)],
                             rows.at[pl.ds(k, CS)], sr[p])

    def drain_reads(g, p):
        for i in range(GK2):
            off = (g * GK2 + i) * CS
            k = (p * GK2 + i) * CS
            pltpu.make_async_copy(msg_hbm.at[pl.ds(ebase + off, CS)],
                                  rows.at[pl.ds(k, CS)], sr[p]).wait()

    def issue_adds(g, p):
        for i in range(GK2):
            j = g * GK2 + i
            k = (p * GK2 + i) * CS
            pltpu.async_copy(rows.at[pl.ds(k, CS)],
                             agg_sh.at[idx_all.at[j]], sa[p], add=True)

    def drain_adds(g, p):
        for i in range(GK2):
            j = g * GK2 + i
            k = (p * GK2 + i) * CS
            pltpu.make_async_copy(rows.at[pl.ds(k, CS)],
                                  agg_sh.at[idx_all.at[j]], sa[p]).wait()

    issue_reads(0, 0)

    def body(gg, carry):
        for p in (0, 1):
            g = 2 * gg + p

            @pl.when(g >= 2)
            def _():
                drain_adds(g - 2, p)

            @pl.when(g >= 1)
            def _():
                issue_reads(g, p)

            @pl.when(g >= 1)
            def _():
                drain_reads(g - 1, 1 - p)
                issue_adds(g - 1, 1 - p)

        return carry

    lax.fori_loop(0, NG2 // 2, body, 0)
    drain_reads(NG2 - 1, 1)
    issue_adds(NG2 - 1, 1)
    drain_adds(NG2 - 2, 0)
    drain_adds(NG2 - 1, 1)
    plsc.subcore_barrier()

    for m in range(NROWS_T // NB2):
        r = row0 + m * NB2
        pltpu.sync_copy(agg_sh.at[pl.ds(r, NB2)], rows)
        pltpu.sync_copy(rows, out_hbm.at[cid].at[pl.ds(r, NB2)])


BE = 2560


def _edge_body(gs, gd, m1d, bm1, w2, bm2, msg):
    xs, a_s = _unpack(gs[...])
    xd, b_d = _unpack(gd[...])
    diff = jnp.abs(xs - xd).astype(jnp.bfloat16)
    pre = (jnp.dot(diff, m1d[...], preferred_element_type=jnp.float32)
           + a_s + b_d + bm1[...])
    h = jnp.maximum(pre, 0.0)
    z = jnp.sum(h * w2[...], axis=1, keepdims=True) + bm2[...]
    ew = 1.0 / (1.0 + jnp.exp(-z))
    msg[...] = xs * ew


def _edge_mlp(gs, gd, m1d, bm1, w2, bm2):
    full = lambda shp: pl.BlockSpec(shp, lambda i: (0,) * len(shp))
    return pl.pallas_call(
        _edge_body,
        grid=(E // BE,),
        in_specs=[
            pl.BlockSpec((BE, D), lambda i: (i, 0)),
            pl.BlockSpec((BE, D), lambda i: (i, 0)),
            full((D, D)),
            full((1, D)), full((1, D)), full((1, 1)),
        ],
        out_specs=pl.BlockSpec((BE, D), lambda i: (i, 0)),
        out_shape=jax.ShapeDtypeStruct((E, D), jnp.float32),
    )(gs, gd, m1d, bm1, w2, bm2)


BN = 2000


def _post_body(a0, a1, xb, wrel, wroot, brel, out):
    agg = a0[...] + a1[...]
    out[...] = (jnp.dot(agg, wrel[...], preferred_element_type=jnp.float32)
                + jnp.dot(xb[...], wroot[...],
                          preferred_element_type=jnp.float32)
                + brel[...])


def _post(a0, a1, x, wrel, wroot, brel):
    full = lambda shp: pl.BlockSpec(shp, lambda i: (0,) * len(shp))
    return pl.pallas_call(
        _post_body,
        grid=(N // BN,),
        in_specs=[
            pl.BlockSpec((BN, D), lambda i: (i, 0)),
            pl.BlockSpec((BN, D), lambda i: (i, 0)),
            pl.BlockSpec((BN, D), lambda i: (i, 0)),
            full((D, D)), full((D, D)), full((1, D)),
        ],
        out_specs=pl.BlockSpec((BN, D), lambda i: (i, 0)),
        out_shape=jax.ShapeDtypeStruct((N, D), jnp.float32),
    )(a0, a1, x, wrel, wroot, brel)


def kernel(x, edge_index, W_pe, b_pe, W_ce, b_ce, W_m1, b_m1, W_m2, b_m2,
           W_rel, b_rel, W_root):
    src = edge_index[0]
    dst = edge_index[1]
    dst3 = dst.reshape(NW, NCH, CS)
    bf = jnp.bfloat16

    ts, td = _pre(x, W_pe, W_ce, W_m1[:D], W_m1[D:2 * D],
                  b_pe.reshape(1, D), b_ce.reshape(1, D))

    gs, gd = _sc_gather(ts, td, src, dst)

    msg = _edge_mlp(gs, gd, W_m1[2 * D:].astype(bf),
                    b_m1.reshape(1, D), W_m2.reshape(1, D),
                    b_m2.reshape(1, 1))

    aggs = _sc_scatter(msg, dst3)

    return _post(aggs[0], aggs[1], x, W_rel, W_root, b_rel.reshape(1, D))


# speedup vs baseline: 4.5589x; 1.1296x over previous
import functools

import jax
import jax.numpy as jnp
from jax import lax
from jax.experimental import pallas as pl
from jax.experimental.pallas import tpu as pltpu
from jax.experimental.pallas import tpu_sc as plsc

N = 10000
E = 320000
D = 128

NC = 2
NS = 16
NW = NC * NS
HALVES = 2
EC = E // HALVES
PER_TILE = EC // NW
CS = 40
NCH = PER_TILE // CS
GK = 5
NG = NCH // GK
N_PAD = 10240
NROWS_T = N_PAD // NS

_mesh = plsc.VectorSubcoreMesh(
    core_axis_name="c", subcore_axis_name="s", num_cores=NC, num_subcores=NS)


def _pipeline(ng, issue_a, drain_a, issue_b, drain_b):
    issue_a(0, 0)

    def body(gg, carry):
        for p in (0, 1):
            g = 2 * gg + p

            @pl.when(g >= 2)
            def _():
                drain_b(g - 2, p)

            @pl.when(g >= 1)
            def _():
                issue_a(g, p)

            @pl.when(g >= 1)
            def _():
                drain_a(g - 1, 1 - p)
                issue_b(g - 1, 1 - p)

        return carry

    lax.fori_loop(0, ng // 2, body, 0)
    if ng % 2:
        drain_b(ng - 3, 0)
        issue_a(ng - 1, 0)
        drain_a(ng - 2, 1)
        issue_b(ng - 2, 1)
        drain_a(ng - 1, 0)
        issue_b(ng - 1, 0)
        drain_b(ng - 2, 1)
        drain_b(ng - 1, 0)
    else:
        drain_a(ng - 1, 1)
        issue_b(ng - 1, 1)
        drain_b(ng - 2, 0)
        drain_b(ng - 1, 1)


def _pack(xf, af):
    xu = lax.bitcast_convert_type(xf, jnp.uint32) >> 16
    au = lax.bitcast_convert_type(af, jnp.uint32) & jnp.uint32(0xFFFF0000)
    return lax.bitcast_convert_type(xu | au, jnp.int32)


def _unpack(w):
    wu = lax.bitcast_convert_type(w, jnp.uint32)
    lo = lax.bitcast_convert_type(wu << 16, jnp.float32)
    hi = lax.bitcast_convert_type(wu & jnp.uint32(0xFFFF0000), jnp.float32)
    return lo, hi


BP = 2000


def _pre_body(xb, wpe, wce, m1p, m1c, bpe, bce, ts, td):
    xv = xb[...]
    t1 = jnp.dot(xv, wpe[...], preferred_element_type=jnp.float32) + bpe[...]
    t1 = jnp.where(t1 >= 0, t1, 0.01 * t1)
    a = jnp.dot(t1, m1p[...], preferred_element_type=jnp.float32)
    t2 = jnp.dot(xv, wce[...], preferred_element_type=jnp.float32) + bce[...]
    t2 = jnp.where(t2 >= 0, t2, 0.01 * t2)
    b = jnp.dot(t2, m1c[...], preferred_element_type=jnp.float32)
    xr = xv.astype(jnp.bfloat16).astype(jnp.float32)
    ts[...] = _pack(xr, a.astype(jnp.bfloat16).astype(jnp.float32))
    td[...] = _pack(xr, b.astype(jnp.bfloat16).astype(jnp.float32))


def _pre(x, wpe, wce, m1p, m1c, bpe, bce):
    full = lambda shp: pl.BlockSpec(shp, lambda i: (0,) * len(shp))
    return pl.pallas_call(
        _pre_body,
        grid=(N // BP,),
        in_specs=[
            pl.BlockSpec((BP, D), lambda i: (i, 0)),
            full((D, D)), full((D, D)), full((D, D)), full((D, D)),
            full((1, D)), full((1, D)),
        ],
        out_specs=[pl.BlockSpec((BP, D), lambda i: (i, 0)),
                   pl.BlockSpec((BP, D), lambda i: (i, 0))],
        out_shape=[jax.ShapeDtypeStruct((N, D), jnp.int32),
                   jax.ShapeDtypeStruct((N, D), jnp.int32)],
    )(x, wpe, wce, m1p, m1c, bpe, bce)


@functools.partial(
    pl.kernel,
    out_type=[jax.ShapeDtypeStruct((EC, D), jnp.int32),
              jax.ShapeDtypeStruct((EC, D), jnp.int32)],
    mesh=_mesh,
    scratch_types=[
        pltpu.VMEM((PER_TILE,), jnp.int32),
        pltpu.VMEM((PER_TILE,), jnp.int32),
        pltpu.VMEM((2 * GK, CS, D), jnp.int32),
        pltpu.VMEM((2 * GK, CS, D), jnp.int32),
    ] + [pltpu.SemaphoreType.DMA] * 8,
)
def _sc_gather(ts_hbm, td_hbm, src_hbm, dst_hbm, gs_hbm, gd_hbm,
               idx_s, idx_d, buf_s, buf_d,
               sg_s0, sg_s1, sg_d0, sg_d1, sw_s0, sw_s1, sw_d0, sw_d1):
    cid = lax.axis_index("c")
    sid = lax.axis_index("s")
    wid = cid * NS + sid
    base = wid * PER_TILE
    pltpu.sync_copy(src_hbm.at[pl.ds(base, PER_TILE)], idx_s)
    pltpu.sync_copy(dst_hbm.at[pl.ds(base, PER_TILE)], idx_d)
    sg = ((sg_s0, sg_d0), (sg_s1, sg_d1))
    sw = ((sw_s0, sw_d0), (sw_s1, sw_d1))

    def issue_gathers(g, p):
        for i in range(GK):
            off = (g * GK + i) * CS
            k = p * GK + i
            pltpu.async_copy(ts_hbm.at[idx_s.at[pl.ds(off, CS)]],
                             buf_s.at[k], sg[p][0])
            pltpu.async_copy(td_hbm.at[idx_d.at[pl.ds(off, CS)]],
                             buf_d.at[k], sg[p][1])

    def drain_gathers(g, p):
        for i in range(GK):
            off = (g * GK + i) * CS
            k = p * GK + i
            pltpu.make_async_copy(ts_hbm.at[idx_s.at[pl.ds(off, CS)]],
                                  buf_s.at[k], sg[p][0]).wait()
            pltpu.make_async_copy(td_hbm.at[idx_d.at[pl.ds(off, CS)]],
                                  buf_d.at[k], sg[p][1]).wait()

    def issue_writes(g, p):
        for i in range(GK):
            off = (g * GK + i) * CS
            k = p * GK + i
            pltpu.async_copy(buf_s.at[k], gs_hbm.at[pl.ds(base + off, CS)],
                             sw[p][0])
            pltpu.async_copy(buf_d.at[k], gd_hbm.at[pl.ds(base + off, CS)],
                             sw[p][1])

    def drain_writes(g, p):
        for i in range(GK):
            off = (g * GK + i) * CS
            k = p * GK + i
            pltpu.make_async_copy(buf_s.at[k],
                                  gs_hbm.at[pl.ds(base + off, CS)],
                                  sw[p][0]).wait()
            pltpu.make_async_copy(buf_d.at[k],
                                  gd_hbm.at[pl.ds(base + off, CS)],
                                  sw[p][1]).wait()

    _pipeline(NG, issue_gathers, drain_gathers, issue_writes, drain_writes)


GK2 = 1
NB2 = 2 * GK2 * CS
NG2 = NCH // GK2


@functools.partial(
    pl.kernel,
    out_type=jax.ShapeDtypeStruct((NC, N_PAD, D), jnp.float32),
    mesh=_mesh,
    scratch_types=[
        pltpu.VMEM((NCH, CS), jnp.int32),
        pltpu.VMEM((NB2, D), jnp.float32),
        pltpu.VMEM_SHARED((N_PAD, D), jnp.float32),
    ] + [pltpu.SemaphoreType.DMA] * 4,
)
def _sc_scatter(msg_hbm, dst3_hbm, out_hbm, idx_all, rows, agg_sh,
                sr0, sr1, sa0, sa1):
    cid = lax.axis_index("c")
    sid = lax.axis_index("s")
    wid = cid * NS + sid

    def zb(t, carry):
        i = t // (D // 16)
        k = t % (D // 16)
        rows[i, pl.ds(k * 16, 16)] = jnp.zeros((16,), jnp.float32)
        return carry

    lax.fori_loop(0, NB2 * (D // 16), zb, 0)
    row0 = sid * NROWS_T
    for m in range(NROWS_T // NB2):
        pltpu.sync_copy(rows, agg_sh.at[pl.ds(row0 + m * NB2, NB2)])
    plsc.subcore_barrier()

    ebase = wid * PER_TILE
    pltpu.sync_copy(dst3_hbm.at[wid], idx_all)
    sr = (sr0, sr1)
    sa = (sa0, sa1)

    def issue_reads(g, p):
        for i in range(GK2):
            off = (g * GK2 + i) * CS
            k = (p * GK2 + i) * CS
            pltpu.async_copy(msg_hbm.at[pl.ds(ebase + off, CS)],
                             rows.at[pl.ds(k, CS)], sr[p])

    def drain_reads(g, p):
        for i in range(GK2):
            off = (g * GK2 + i) * CS
            k = (p * GK2 + i) * CS
            pltpu.make_async_copy(msg_hbm.at[pl.ds(ebase + off, CS)],
                                  rows.at[pl.ds(k, CS)], sr[p]).wait()

    def issue_adds(g, p):
        for i in range(GK2):
            j = g * GK2 + i
            k = (p * GK2 + i) * CS
            pltpu.async_copy(rows.at[pl.ds(k, CS)],
                             agg_sh.at[idx_all.at[j]], sa[p], add=True)

    def drain_adds(g, p):
        for i in range(GK2):
            j = g * GK2 + i
            k = (p * GK2 + i) * CS
            pltpu.make_async_copy(rows.at[pl.ds(k, CS)],
                                  agg_sh.at[idx_all.at[j]], sa[p]).wait()

    _pipeline(NG2, issue_reads, drain_reads, issue_adds, drain_adds)
    plsc.subcore_barrier()

    for m in range(NROWS_T // NB2):
        r = row0 + m * NB2
        pltpu.sync_copy(agg_sh.at[pl.ds(r, NB2)], rows)
        pltpu.sync_copy(rows, out_hbm.at[cid].at[pl.ds(r, NB2)])


BE = 3200


def _edge_body(gs, gd, m1d, bm1, w2, bm2, msg):
    xs, a_s = _unpack(gs[...])
    xd, b_d = _unpack(gd[...])
    diff = jnp.abs(xs - xd).astype(jnp.bfloat16)
    pre = (jnp.dot(diff, m1d[...], preferred_element_type=jnp.float32)
           + a_s + b_d + bm1[...])
    h = jnp.maximum(pre, 0.0)
    z = jnp.sum(h * w2[...], axis=1, keepdims=True) + bm2[...]
    ew = 1.0 / (1.0 + jnp.exp(-z))
    msg[...] = xs * ew


def _edge_mlp(gs, gd, m1d, bm1, w2, bm2):
    full = lambda shp: pl.BlockSpec(shp, lambda i: (0,) * len(shp))
    return pl.pallas_call(
        _edge_body,
        grid=(EC // BE,),
        in_specs=[
            pl.BlockSpec((BE, D), lambda i: (i, 0)),
            pl.BlockSpec((BE, D), lambda i: (i, 0)),
            full((D, D)),
            full((1, D)), full((1, D)), full((1, 1)),
        ],
        out_specs=pl.BlockSpec((BE, D), lambda i: (i, 0)),
        out_shape=jax.ShapeDtypeStruct((EC, D), jnp.float32),
    )(gs, gd, m1d, bm1, w2, bm2)


BN = 2000


def _post_body(a0, a1, a2, a3, xb, wrel, wroot, brel, out):
    agg = (a0[...] + a1[...]) + (a2[...] + a3[...])
    out[...] = (jnp.dot(agg, wrel[...], preferred_element_type=jnp.float32)
                + jnp.dot(xb[...], wroot[...],
                          preferred_element_type=jnp.float32)
                + brel[...])


def _post(a0, a1, a2, a3, x, wrel, wroot, brel):
    full = lambda shp: pl.BlockSpec(shp, lambda i: (0,) * len(shp))
    blk = pl.BlockSpec((BN, D), lambda i: (i, 0))
    return pl.pallas_call(
        _post_body,
        grid=(N // BN,),
        in_specs=[blk, blk, blk, blk, blk,
                  full((D, D)), full((D, D)), full((1, D))],
        out_specs=blk,
        out_shape=jax.ShapeDtypeStruct((N, D), jnp.float32),
    )(a0, a1, a2, a3, x, wrel, wroot, brel)


def kernel(x, edge_index, W_pe, b_pe, W_ce, b_ce, W_m1, b_m1, W_m2, b_m2,
           W_rel, b_rel, W_root):
    src = edge_index[0]
    dst = edge_index[1]
    dst4 = dst.reshape(HALVES, NW, NCH, CS)
    bf = jnp.bfloat16

    ts, td = _pre(x, W_pe, W_ce, W_m1[:D], W_m1[D:2 * D],
                  b_pe.reshape(1, D), b_ce.reshape(1, D))

    m1d = W_m1[2 * D:].astype(bf)
    bm1 = b_m1.reshape(1, D)
    w2 = W_m2.reshape(1, D)
    bm2 = b_m2.reshape(1, 1)

    parts = []
    for c in range(HALVES):
        gs, gd = _sc_gather(ts, td, src[c * EC:(c + 1) * EC],
                            dst[c * EC:(c + 1) * EC])
        msg = _edge_mlp(gs, gd, m1d, bm1, w2, bm2)
        parts.append(_sc_scatter(msg, dst4[c]))

    return _post(parts[0][0], parts[0][1], parts[1][0], parts[1][1],
                 x, W_rel, W_root, b_rel.reshape(1, D))
